# baseline (speedup 1.0000x reference)
import functools

import jax
import jax.numpy as jnp
import numpy as np
from jax import lax
from jax.experimental import pallas as pl
from jax.experimental.pallas import tpu as pltpu
from jax.experimental.pallas import tpu_sc as plsc

B = 16384
D = 64
NC = 2
NS = 16
NW = NC * NS
BPW = B // NW
CHUNK = 128
NCH = BPW // CHUNK

_INV_SQRT = float(1.0 / np.sqrt(1.0 + 1e-5))


def _pack_table(table):
    n = table.shape[0]
    t3 = table.reshape(n // 4, 2, 128)
    a = lax.bitcast_convert_type(t3[:, 0, :].astype(jnp.bfloat16), jnp.uint16)
    b = lax.bitcast_convert_type(t3[:, 1, :].astype(jnp.bfloat16), jnp.uint16)
    packed = (a.astype(jnp.uint32) << 16) | b.astype(jnp.uint32)
    return lax.bitcast_convert_type(packed, jnp.int32)


@functools.cache
def _make_sc_gather(nrow_u, nrow_i):
    mesh = plsc.VectorSubcoreMesh(core_axis_name="c", subcore_axis_name="s")

    @functools.partial(
        pl.kernel,
        mesh=mesh,
        out_type=[
            jax.ShapeDtypeStruct((B, 128), jnp.int32),
            jax.ShapeDtypeStruct((B, 128), jnp.int32),
        ],
        scratch_types=[
            pltpu.VMEM((NCH, CHUNK), jnp.int32),
            pltpu.VMEM((NCH, CHUNK), jnp.int32),
            pltpu.VMEM((BPW, 128), jnp.int32),
            pltpu.SemaphoreType.DMA,
        ],
    )
    def _sc_gather(uq_hbm, iq_hbm, ut_hbm, it_hbm, ue_hbm, ie_hbm,
                   uq, iq, rows, sem):
        wid = lax.axis_index("s") * NC + lax.axis_index("c")
        base = wid * BPW
        pltpu.sync_copy(uq_hbm.at[wid], uq)
        pltpu.sync_copy(iq_hbm.at[wid], iq)
        for tbl, idx, out in ((ut_hbm, uq, ue_hbm), (it_hbm, iq, ie_hbm)):
            copies = [
                pltpu.async_copy(tbl.at[idx.at[j]],
                                 rows.at[pl.ds(j * CHUNK, CHUNK)], sem)
                for j in range(NCH)
            ]
            for c in copies:
                c.wait()
            pltpu.sync_copy(rows, out.at[pl.ds(base, BPW)])

    return _sc_gather


BLK = 2048
GRID = B // BLK


def _unpack_select(v32, ids):
    hi = lax.bitcast_convert_type(
        lax.shift_right_logical(v32, 16).astype(jnp.uint16), jnp.bfloat16
    ).astype(jnp.float32)
    lo = lax.bitcast_convert_type(
        (v32 & 0xFFFF).astype(jnp.uint16), jnp.bfloat16
    ).astype(jnp.float32)
    s_bit = ((ids >> 1) & 1)[:, None]
    row128 = jnp.where(s_bit == 1, lo, hi)
    odd = (ids & 1)[:, None]
    return jnp.where(odd == 1, row128[:, D:], row128[:, :D])


def _mlp_body(ue_ref, ie_ref, uid_ref, iid_ref,
              w0_ref, b0_ref, g0_ref, bt0_ref,
              w1_ref, b1_ref, g1_ref, bt1_ref,
              w2_ref, b2_ref, g2_ref, bt2_ref,
              wo_ref, bo_ref, out_ref):
    xu = _unpack_select(ue_ref[...], uid_ref[0, 0, :])
    xi = _unpack_select(ie_ref[...], iid_ref[0, 0, :])
    x = (jnp.dot(xu, w0_ref[0:D, :], preferred_element_type=jnp.float32)
         + jnp.dot(xi, w0_ref[D:2 * D, :], preferred_element_type=jnp.float32)
         + b0_ref[...])
    x = jnp.maximum(x, 0.0) * (g0_ref[...] * _INV_SQRT) + bt0_ref[...]
    x = jnp.dot(x, w1_ref[...], preferred_element_type=jnp.float32) + b1_ref[...]
    x = jnp.maximum(x, 0.0) * (g1_ref[...] * _INV_SQRT) + bt1_ref[...]
    x = jnp.dot(x, w2_ref[...], preferred_element_type=jnp.float32) + b2_ref[...]
    x = jnp.maximum(x, 0.0) * (g2_ref[...] * _INV_SQRT) + bt2_ref[...]
    z = jnp.sum(x * wo_ref[...], axis=1) + bo_ref[0, 0]
    out_ref[...] = 1.0 / (1.0 + jnp.exp(-z))


def _full(shape):
    return pl.BlockSpec(shape, lambda i: (0,) * len(shape))


def _mlp_call(ue32, ie32, user_ids, item_ids, weights):
    (W0, b0, gamma0, beta0, W1, b1, gamma1, beta1,
     W2, b2, gamma2, beta2, Wo, bo) = weights
    ins = [ue32, ie32,
           user_ids.reshape(GRID, 1, BLK), item_ids.reshape(GRID, 1, BLK)]
    in_specs = [
        pl.BlockSpec((BLK, 128), lambda i: (i, 0)),
        pl.BlockSpec((BLK, 128), lambda i: (i, 0)),
        pl.BlockSpec((1, 1, BLK), lambda i: (i, 0, 0)),
        pl.BlockSpec((1, 1, BLK), lambda i: (i, 0, 0)),
    ]
    for (W, b, g, bt) in ((W0, b0, gamma0, beta0), (W1, b1, gamma1, beta1),
                          (W2, b2, gamma2, beta2)):
        h = W.shape[1]
        ins += [W, b.reshape(1, h), g.reshape(1, h), bt.reshape(1, h)]
        in_specs += [_full(W.shape), _full((1, h)), _full((1, h)), _full((1, h))]
    ins += [Wo.reshape(1, Wo.shape[0]), bo.reshape(1, 1)]
    in_specs += [_full((1, Wo.shape[0])),
                 pl.BlockSpec(memory_space=pltpu.SMEM)]
    return pl.pallas_call(
        _mlp_body,
        grid=(GRID,),
        in_specs=in_specs,
        out_specs=pl.BlockSpec((BLK,), lambda i: (i,)),
        out_shape=jax.ShapeDtypeStruct((B,), jnp.float32),
    )(*ins)


def kernel(user_ids, item_ids, user_table, item_table,
           W0, b0, gamma0, beta0,
           W1, b1, gamma1, beta1,
           W2, b2, gamma2, beta2,
           Wo, bo):
    user_ids = user_ids.astype(jnp.int32)
    item_ids = item_ids.astype(jnp.int32)
    ut32 = _pack_table(user_table)
    it32 = _pack_table(item_table)
    uq3 = (user_ids >> 2).reshape(NW, NCH, CHUNK)
    iq3 = (item_ids >> 2).reshape(NW, NCH, CHUNK)
    ue32, ie32 = _make_sc_gather(ut32.shape[0], it32.shape[0])(
        uq3, iq3, ut32, it32)
    return _mlp_call(ue32, ie32, user_ids, item_ids,
                     (W0, b0, gamma0, beta0, W1, b1, gamma1, beta1,
                      W2, b2, gamma2, beta2, Wo, bo))


# speedup vs baseline: 1.8945x; 1.8945x over previous
import functools

import jax
import jax.numpy as jnp
import numpy as np
from jax import lax
from jax.experimental import pallas as pl
from jax.experimental.pallas import tpu as pltpu
from jax.experimental.pallas import tpu_sc as plsc

B = 16384
D = 64
NC = 2
NS = 16
NW = NC * NS
BPW = B // NW
CHUNK = 128
NCH = BPW // CHUNK

_INV_SQRT = float(1.0 / np.sqrt(1.0 + 1e-5))


@functools.cache
def _make_sc_gather():
    mesh = plsc.VectorSubcoreMesh(core_axis_name="c", subcore_axis_name="s")

    @functools.partial(
        pl.kernel,
        mesh=mesh,
        out_type=[
            jax.ShapeDtypeStruct((B, 128), jnp.float32),
            jax.ShapeDtypeStruct((B, 128), jnp.float32),
        ],
        scratch_types=[
            pltpu.VMEM((NCH, CHUNK), jnp.int32),
            pltpu.VMEM((NCH, CHUNK), jnp.int32),
            pltpu.VMEM((BPW, 128), jnp.float32),
            pltpu.SemaphoreType.DMA,
        ],
    )
    def _sc_gather(uq_hbm, iq_hbm, ut_hbm, it_hbm, ue_hbm, ie_hbm,
                   uq, iq, rows, sem):
        wid = lax.axis_index("s") * NC + lax.axis_index("c")
        base = wid * BPW
        pltpu.sync_copy(uq_hbm.at[wid], uq)
        pltpu.sync_copy(iq_hbm.at[wid], iq)
        for tbl, idx, out in ((ut_hbm, uq, ue_hbm), (it_hbm, iq, ie_hbm)):
            copies = [
                pltpu.async_copy(tbl.at[idx.at[j]],
                                 rows.at[pl.ds(j * CHUNK, CHUNK)], sem)
                for j in range(NCH)
            ]
            for c in copies:
                c.wait()
            pltpu.sync_copy(rows, out.at[pl.ds(base, BPW)])

    return _sc_gather


KBLK = 1024
TGRID = 1_000_000 // (2 * KBLK)
NPACK = TGRID * KBLK
TSTART = 2 * KBLK * TGRID
TAIL = 1_000_000 - TSTART


def _tr_body(xa_ref, xb_ref, i_ref, out_ref):
    ident = i_ref[...]
    dn = (((0,), (0,)), ((), ()))
    out_ref[:, 0:D] = lax.dot_general(
        xa_ref[...], ident, dn, preferred_element_type=jnp.float32)
    out_ref[:, D:2 * D] = lax.dot_general(
        xb_ref[...], ident, dn, preferred_element_type=jnp.float32)


def _transpose_pack(table):
    table_t = table.T
    return pl.pallas_call(
        _tr_body,
        grid=(TGRID,),
        in_specs=[
            pl.BlockSpec((D, KBLK), lambda i: (0, 2 * i)),
            pl.BlockSpec((D, KBLK), lambda i: (0, 2 * i + 1)),
            pl.BlockSpec((D, D), lambda i: (0, 0)),
        ],
        out_specs=pl.BlockSpec((KBLK, 128), lambda i: (i, 0)),
        out_shape=jax.ShapeDtypeStruct((NPACK, 128), jnp.float32),
    )(table_t, table_t, jnp.eye(D, dtype=jnp.float32))


def _packed_row(ids):
    q = ((ids >> 11) << 10) + (ids & 1023)
    return jnp.minimum(q, NPACK - 1)


BLK = 2048
GRID = B // BLK


def _tail_fix(x, ids2d, tail_ref):
    onehot = (lax.broadcasted_iota(jnp.int32, (BLK, TAIL), 1)
              == (ids2d - TSTART)).astype(jnp.float32)
    xfix = lax.dot_general(onehot, tail_ref[...], (((1,), (1,)), ((), ())),
                           preferred_element_type=jnp.float32)
    return jnp.where(ids2d >= TSTART, xfix, x)


def _mlp_body(ue_ref, ie_ref, uid_ref, iid_ref, tailu_ref, taili_ref,
              w0_ref, b0_ref, g0_ref, bt0_ref,
              w1_ref, b1_ref, g1_ref, bt1_ref,
              w2_ref, b2_ref, g2_ref, bt2_ref,
              wo_ref, bo_ref, out_ref):
    uid2 = uid_ref[0]
    iid2 = iid_ref[0]
    uhi = ((uid2 >> 10) & 1) == 1
    ihi = ((iid2 >> 10) & 1) == 1
    xu = jnp.where(uhi, ue_ref[:, D:], ue_ref[:, :D])
    xi = jnp.where(ihi, ie_ref[:, D:], ie_ref[:, :D])
    xu = _tail_fix(xu, uid2, tailu_ref)
    xi = _tail_fix(xi, iid2, taili_ref)
    x = (jnp.dot(xu, w0_ref[0:D, :], preferred_element_type=jnp.float32)
         + jnp.dot(xi, w0_ref[D:2 * D, :], preferred_element_type=jnp.float32)
         + b0_ref[...])
    x = jnp.maximum(x, 0.0) * (g0_ref[...] * _INV_SQRT) + bt0_ref[...]
    x = jnp.dot(x, w1_ref[...], preferred_element_type=jnp.float32) + b1_ref[...]
    x = jnp.maximum(x, 0.0) * (g1_ref[...] * _INV_SQRT) + bt1_ref[...]
    x = jnp.dot(x, w2_ref[...], preferred_element_type=jnp.float32) + b2_ref[...]
    x = jnp.maximum(x, 0.0) * (g2_ref[...] * _INV_SQRT) + bt2_ref[...]
    z = jnp.sum(x * wo_ref[...], axis=1) + bo_ref[0, 0]
    out_ref[...] = 1.0 / (1.0 + jnp.exp(-z))


def _full(shape):
    return pl.BlockSpec(shape, lambda i: (0,) * len(shape))


def _mlp_call(ue, ie, user_ids, item_ids, tailu, taili, weights):
    (W0, b0, gamma0, beta0, W1, b1, gamma1, beta1,
     W2, b2, gamma2, beta2, Wo, bo) = weights
    ins = [ue, ie,
           user_ids.reshape(GRID, BLK, 1), item_ids.reshape(GRID, BLK, 1),
           tailu, taili]
    in_specs = [
        pl.BlockSpec((BLK, 128), lambda i: (i, 0)),
        pl.BlockSpec((BLK, 128), lambda i: (i, 0)),
        pl.BlockSpec((1, BLK, 1), lambda i: (i, 0, 0)),
        pl.BlockSpec((1, BLK, 1), lambda i: (i, 0, 0)),
        _full((D, TAIL)),
        _full((D, TAIL)),
    ]
    for (W, b, g, bt) in ((W0, b0, gamma0, beta0), (W1, b1, gamma1, beta1),
                          (W2, b2, gamma2, beta2)):
        h = W.shape[1]
        ins += [W, b.reshape(1, h), g.reshape(1, h), bt.reshape(1, h)]
        in_specs += [_full(W.shape), _full((1, h)), _full((1, h)), _full((1, h))]
    ins += [Wo.reshape(1, Wo.shape[0]), bo.reshape(1, 1)]
    in_specs += [_full((1, Wo.shape[0])),
                 pl.BlockSpec(memory_space=pltpu.SMEM)]
    return pl.pallas_call(
        _mlp_body,
        grid=(GRID,),
        in_specs=in_specs,
        out_specs=pl.BlockSpec((BLK,), lambda i: (i,)),
        out_shape=jax.ShapeDtypeStruct((B,), jnp.float32),
    )(*ins)


def kernel(user_ids, item_ids, user_table, item_table,
           W0, b0, gamma0, beta0,
           W1, b1, gamma1, beta1,
           W2, b2, gamma2, beta2,
           Wo, bo):
    user_ids = user_ids.astype(jnp.int32)
    item_ids = item_ids.astype(jnp.int32)
    ut2 = _transpose_pack(user_table)
    it2 = _transpose_pack(item_table)
    uq3 = _packed_row(user_ids).reshape(NW, NCH, CHUNK)
    iq3 = _packed_row(item_ids).reshape(NW, NCH, CHUNK)
    ue, ie = _make_sc_gather()(uq3, iq3, ut2, it2)
    tailu = user_table.T[:, TSTART:]
    taili = item_table.T[:, TSTART:]
    return _mlp_call(ue, ie, user_ids, item_ids, tailu, taili,
                     (W0, b0, gamma0, beta0, W1, b1, gamma1, beta1,
                      W2, b2, gamma2, beta2, Wo, bo))
